# baseline (speedup 1.0000x reference)
import functools

import jax
import jax.numpy as jnp
from jax import lax
from jax.experimental import pallas as pl
from jax.experimental.pallas import tpu as pltpu
from jax.experimental.pallas import tpu_sc as plsc

_B = 8
_N = 8192
_C = 128
_NPOINT = 2048


def _fps_body(xs_ref, ys_ref, zs_ref, idx_ref, cx_ref, cy_ref, cz_ref,
              dist_ref):
    lane = lax.broadcasted_iota(jnp.int32, (_B, 128), 1)
    rowbase = lax.broadcasted_iota(jnp.int32, (_B, 1), 0) * _N
    dist_ref[...] = jnp.full((_B, _N), 1e10, jnp.float32)
    nchunk = _N // 128

    def body(l, carry):
        far, cx, cy, cz, ai, ax, ay, az = carry
        lm = lane == l
        ai = jnp.where(lm, far + rowbase, ai)
        ax = jnp.where(lm, cx, ax)
        ay = jnp.where(lm, cy, ay)
        az = jnp.where(lm, cz, az)
        nset = 4
        cps = nchunk // nset
        ninf = jnp.full((_B, 128), -jnp.inf, jnp.float32)
        zi128 = jnp.zeros((_B, 128), jnp.int32)
        zf128 = jnp.zeros((_B, 128), jnp.float32)

        def cbody(k, sc):
            sets = [list(sc[5 * s:5 * s + 5]) for s in range(nset)]
            off0 = pl.multiple_of(k * 128, 128)
            for s in range(nset):
                sb, si, sx, sy, sz = sets[s]
                sl = pl.ds(off0 + s * (cps * 128), 128)
                xc = xs_ref[:, sl]
                yc = ys_ref[:, sl]
                zc = zs_ref[:, sl]
                d = ((xc - cx) ** 2 + (zc - cz) ** 2) + (yc - cy) ** 2
                dn = jnp.minimum(dist_ref[:, sl], d)
                dist_ref[:, sl] = dn
                g = dn > sb
                sets[s] = [jnp.where(g, dn, sb),
                           jnp.where(g, s * cps + k, si),
                           jnp.where(g, xc, sx),
                           jnp.where(g, yc, sy),
                           jnp.where(g, zc, sz)]
            return tuple(v for st in sets for v in st)

        init = tuple(v for _ in range(nset)
                     for v in (ninf, zi128, zf128, zf128, zf128))
        res = lax.fori_loop(0, cps, cbody, init, unroll=2)
        best, besti, bx, by, bz = res[0:5]
        for s in range(1, nset):
            sb, si, sx, sy, sz = res[5 * s:5 * s + 5]
            g = sb > best
            best = jnp.where(g, sb, best)
            besti = jnp.where(g, si, besti)
            bx = jnp.where(g, sx, bx)
            by = jnp.where(g, sy, by)
            bz = jnp.where(g, sz, bz)
        maxv = jnp.max(best, axis=1, keepdims=True)
        cand = jnp.where(best == maxv, besti * 128 + lane, _N)
        far = jnp.min(cand, axis=1, keepdims=True)
        m2 = cand == far
        cx = jnp.sum(jnp.where(m2, bx, 0.0), axis=1, keepdims=True)
        cy = jnp.sum(jnp.where(m2, by, 0.0), axis=1, keepdims=True)
        cz = jnp.sum(jnp.where(m2, bz, 0.0), axis=1, keepdims=True)
        return far, cx, cy, cz, ai, ax, ay, az

    far = jnp.zeros((_B, 1), jnp.int32)
    cx = xs_ref[:, pl.ds(0, 1)]
    cy = ys_ref[:, pl.ds(0, 1)]
    cz = zs_ref[:, pl.ds(0, 1)]
    zi = jnp.zeros((_B, 128), jnp.int32)
    zf = jnp.zeros((_B, 128), jnp.float32)
    for j in range(_NPOINT // 128):
        far, cx, cy, cz, ai, ax, ay, az = lax.fori_loop(
            0, 128, body, (far, cx, cy, cz, zi, zf, zf, zf), unroll=False)
        sl = pl.ds(j * 128, 128)
        idx_ref[:, sl] = ai
        cx_ref[:, sl] = ax
        cy_ref[:, sl] = ay
        cz_ref[:, sl] = az


def _fps_call(xs, ys, zs):
    out_shape = [
        jax.ShapeDtypeStruct((_B, _NPOINT), jnp.int32),
        jax.ShapeDtypeStruct((_B, _NPOINT), jnp.float32),
        jax.ShapeDtypeStruct((_B, _NPOINT), jnp.float32),
        jax.ShapeDtypeStruct((_B, _NPOINT), jnp.float32),
    ]
    return pl.pallas_call(
        _fps_body,
        out_shape=out_shape,
        scratch_shapes=[pltpu.VMEM((_B, _N), jnp.float32)],
    )(xs, ys, zs)


def _sc_gather(table, idx3):
    info = plsc.get_sparse_core_info()
    nw = info.num_cores * info.num_subcores
    rows_per_w = (_B * _NPOINT) // nw
    chunks = rows_per_w // 128
    mesh = plsc.VectorSubcoreMesh(core_axis_name="c", subcore_axis_name="s")

    @functools.partial(
        pl.kernel,
        mesh=mesh,
        out_type=jax.ShapeDtypeStruct((_B * _NPOINT, _C), jnp.float32),
        scratch_types=[
            pltpu.VMEM((chunks, 128), jnp.int32),
            pltpu.VMEM((rows_per_w, _C), jnp.float32),
            pltpu.SemaphoreType.DMA,
        ],
    )
    def k(table_hbm, idx_hbm, out_hbm, idx_v, rows_v, sem):
        wid = lax.axis_index("s") * info.num_cores + lax.axis_index("c")
        base = wid * rows_per_w
        pltpu.sync_copy(idx_hbm.at[wid], idx_v)
        cps = [
            pltpu.async_copy(table_hbm.at[idx_v.at[c]],
                             rows_v.at[pl.ds(c * 128, 128)], sem)
            for c in range(chunks)
        ]
        for cp in cps:
            cp.wait()
        pltpu.sync_copy(rows_v, out_hbm.at[pl.ds(base, rows_per_w)])

    return k(table, idx3)


def kernel(x, x_xyz, n_pints):
    xs = x_xyz[:, :, 0]
    ys = x_xyz[:, :, 1]
    zs = x_xyz[:, :, 2]
    idxg, cx, cy, cz = _fps_call(xs, ys, zs)
    new_xyz = jnp.stack([cx, cy, cz], axis=-1)

    info = plsc.get_sparse_core_info()
    nw = info.num_cores * info.num_subcores
    chunks = (_B * _NPOINT) // nw // 128
    idx3 = idxg.reshape(nw, chunks, 128)
    new_x = _sc_gather(x.reshape(_B * _N, _C), idx3)
    return (new_x.reshape(_B, _NPOINT, _C), new_xyz)


# speedup vs baseline: 1.4241x; 1.4241x over previous
import functools

import jax
import jax.numpy as jnp
from jax import lax
from jax.experimental import pallas as pl
from jax.experimental.pallas import tpu as pltpu
from jax.experimental.pallas import tpu_sc as plsc

_B = 8
_N = 8192
_C = 128
_NPOINT = 2048


def _fps_body(xs_ref, ys_ref, zs_ref, idx_ref, cx_ref, cy_ref, cz_ref,
              dist_ref):
    lane = lax.broadcasted_iota(jnp.int32, (_B, 128), 1)
    rowbase = lax.broadcasted_iota(jnp.int32, (_B, 1), 0) * _N
    dist_ref[...] = jnp.full((_B, _N), 1e10, jnp.float32)
    nchunk = _N // 128

    def body(l, carry):
        far, cx, cy, cz, ai, ax, ay, az = carry
        lm = lane == l
        ai = jnp.where(lm, far + rowbase, ai)
        ax = jnp.where(lm, cx, ax)
        ay = jnp.where(lm, cy, ay)
        az = jnp.where(lm, cz, az)
        nset = 2
        cps = nchunk // nset
        ninf = jnp.full((_B, 128), -jnp.inf, jnp.float32)
        zi128 = jnp.zeros((_B, 128), jnp.int32)
        zf128 = jnp.zeros((_B, 128), jnp.float32)
        sets = [[ninf, zi128, zf128, zf128, zf128] for _ in range(nset)]
        for s in range(nset):
            sb, si, sx, sy, sz = sets[s]
            for k in range(cps):
                c = s * cps + k
                sl = pl.ds(c * 128, 128)
                xc = xs_ref[:, sl]
                yc = ys_ref[:, sl]
                zc = zs_ref[:, sl]
                d = ((xc - cx) ** 2 + (zc - cz) ** 2) + (yc - cy) ** 2
                dn = jnp.minimum(dist_ref[:, sl], d)
                dist_ref[:, sl] = dn
                g = dn > sb
                sb = jnp.where(g, dn, sb)
                si = jnp.where(g, c, si)
                sx = jnp.where(g, xc, sx)
                sy = jnp.where(g, yc, sy)
                sz = jnp.where(g, zc, sz)
            sets[s] = [sb, si, sx, sy, sz]
        best, besti, bx, by, bz = sets[0]
        for s in range(1, nset):
            sb, si, sx, sy, sz = sets[s]
            g = sb > best
            best = jnp.where(g, sb, best)
            besti = jnp.where(g, si, besti)
            bx = jnp.where(g, sx, bx)
            by = jnp.where(g, sy, by)
            bz = jnp.where(g, sz, bz)
        maxv = jnp.max(best, axis=1, keepdims=True)
        cand = jnp.where(best == maxv, besti * 128 + lane, _N)
        far = jnp.min(cand, axis=1, keepdims=True)
        m2 = cand == far
        cx = jnp.sum(jnp.where(m2, bx, 0.0), axis=1, keepdims=True)
        cy = jnp.sum(jnp.where(m2, by, 0.0), axis=1, keepdims=True)
        cz = jnp.sum(jnp.where(m2, bz, 0.0), axis=1, keepdims=True)
        return far, cx, cy, cz, ai, ax, ay, az

    far = jnp.zeros((_B, 1), jnp.int32)
    cx = xs_ref[:, pl.ds(0, 1)]
    cy = ys_ref[:, pl.ds(0, 1)]
    cz = zs_ref[:, pl.ds(0, 1)]
    zi = jnp.zeros((_B, 128), jnp.int32)
    zf = jnp.zeros((_B, 128), jnp.float32)
    for j in range(_NPOINT // 128):
        far, cx, cy, cz, ai, ax, ay, az = lax.fori_loop(
            0, 128, body, (far, cx, cy, cz, zi, zf, zf, zf), unroll=False)
        sl = pl.ds(j * 128, 128)
        idx_ref[:, sl] = ai
        cx_ref[:, sl] = ax
        cy_ref[:, sl] = ay
        cz_ref[:, sl] = az


def _fps_call(xs, ys, zs):
    out_shape = [
        jax.ShapeDtypeStruct((_B, _NPOINT), jnp.int32),
        jax.ShapeDtypeStruct((_B, _NPOINT), jnp.float32),
        jax.ShapeDtypeStruct((_B, _NPOINT), jnp.float32),
        jax.ShapeDtypeStruct((_B, _NPOINT), jnp.float32),
    ]
    return pl.pallas_call(
        _fps_body,
        out_shape=out_shape,
        scratch_shapes=[pltpu.VMEM((_B, _N), jnp.float32)],
    )(xs, ys, zs)


def _sc_gather(table, idx3):
    info = plsc.get_sparse_core_info()
    nw = info.num_cores * info.num_subcores
    rows_per_w = (_B * _NPOINT) // nw
    chunks = rows_per_w // 128
    mesh = plsc.VectorSubcoreMesh(core_axis_name="c", subcore_axis_name="s")

    @functools.partial(
        pl.kernel,
        mesh=mesh,
        out_type=jax.ShapeDtypeStruct((_B * _NPOINT, _C), jnp.float32),
        scratch_types=[
            pltpu.VMEM((chunks, 128), jnp.int32),
            pltpu.VMEM((rows_per_w, _C), jnp.float32),
            pltpu.SemaphoreType.DMA,
        ],
    )
    def k(table_hbm, idx_hbm, out_hbm, idx_v, rows_v, sem):
        wid = lax.axis_index("s") * info.num_cores + lax.axis_index("c")
        base = wid * rows_per_w
        pltpu.sync_copy(idx_hbm.at[wid], idx_v)
        cps = [
            pltpu.async_copy(table_hbm.at[idx_v.at[c]],
                             rows_v.at[pl.ds(c * 128, 128)], sem)
            for c in range(chunks)
        ]
        for cp in cps:
            cp.wait()
        pltpu.sync_copy(rows_v, out_hbm.at[pl.ds(base, rows_per_w)])

    return k(table, idx3)


def kernel(x, x_xyz, n_pints):
    xs = x_xyz[:, :, 0]
    ys = x_xyz[:, :, 1]
    zs = x_xyz[:, :, 2]
    idxg, cx, cy, cz = _fps_call(xs, ys, zs)
    new_xyz = jnp.stack([cx, cy, cz], axis=-1)

    info = plsc.get_sparse_core_info()
    nw = info.num_cores * info.num_subcores
    chunks = (_B * _NPOINT) // nw // 128
    idx3 = idxg.reshape(nw, chunks, 128)
    new_x = _sc_gather(x.reshape(_B * _N, _C), idx3)
    return (new_x.reshape(_B, _NPOINT, _C), new_xyz)


# speedup vs baseline: 1.5534x; 1.0908x over previous
import functools

import jax
import jax.numpy as jnp
from jax import lax
from jax.experimental import pallas as pl
from jax.experimental.pallas import tpu as pltpu
from jax.experimental.pallas import tpu_sc as plsc

_B = 8
_N = 8192
_C = 128
_NPOINT = 2048


def _fps_body(xs_ref, ys_ref, zs_ref, idx_ref, cx_ref, cy_ref, cz_ref,
              dist_ref):
    lane = lax.broadcasted_iota(jnp.int32, (_B, 128), 1)
    rowbase = lax.broadcasted_iota(jnp.int32, (_B, 1), 0) * _N
    dist_ref[...] = jnp.full((_B, _N), 1e10, jnp.float32)
    nchunk = _N // 128

    def body(l, carry):
        far, cx, cy, cz, ai, ax, ay, az = carry
        lm = lane == l
        ai = jnp.where(lm, far + rowbase, ai)
        ax = jnp.where(lm, cx, ax)
        ay = jnp.where(lm, cy, ay)
        az = jnp.where(lm, cz, az)
        nset = 1
        cps = nchunk // nset
        ninf = jnp.full((_B, 128), -jnp.inf, jnp.float32)
        zi128 = jnp.zeros((_B, 128), jnp.int32)
        zf128 = jnp.zeros((_B, 128), jnp.float32)
        sets = [[ninf, zi128, zf128, zf128, zf128] for _ in range(nset)]
        for s in range(nset):
            sb, si, sx, sy, sz = sets[s]
            for k in range(cps):
                c = s * cps + k
                sl = pl.ds(c * 128, 128)
                xc = xs_ref[:, sl]
                yc = ys_ref[:, sl]
                zc = zs_ref[:, sl]
                d = ((xc - cx) ** 2 + (zc - cz) ** 2) + (yc - cy) ** 2
                dn = jnp.minimum(dist_ref[:, sl], d)
                dist_ref[:, sl] = dn
                g = dn > sb
                sb = jnp.maximum(dn, sb)
                si = jnp.where(g, c, si)
                sx = jnp.where(g, xc, sx)
                sy = jnp.where(g, yc, sy)
                sz = jnp.where(g, zc, sz)
            sets[s] = [sb, si, sx, sy, sz]
        best, besti, bx, by, bz = sets[0]
        for s in range(1, nset):
            sb, si, sx, sy, sz = sets[s]
            g = sb > best
            best = jnp.where(g, sb, best)
            besti = jnp.where(g, si, besti)
            bx = jnp.where(g, sx, bx)
            by = jnp.where(g, sy, by)
            bz = jnp.where(g, sz, bz)
        maxv = jnp.max(best, axis=1, keepdims=True)
        cand = jnp.where(best == maxv, besti * 128 + lane, _N)
        far = jnp.min(cand, axis=1, keepdims=True)
        m2 = cand == far
        cx = jnp.sum(jnp.where(m2, bx, 0.0), axis=1, keepdims=True)
        cy = jnp.sum(jnp.where(m2, by, 0.0), axis=1, keepdims=True)
        cz = jnp.sum(jnp.where(m2, bz, 0.0), axis=1, keepdims=True)
        return far, cx, cy, cz, ai, ax, ay, az

    far = jnp.zeros((_B, 1), jnp.int32)
    cx = xs_ref[:, pl.ds(0, 1)]
    cy = ys_ref[:, pl.ds(0, 1)]
    cz = zs_ref[:, pl.ds(0, 1)]
    zi = jnp.zeros((_B, 128), jnp.int32)
    zf = jnp.zeros((_B, 128), jnp.float32)
    for j in range(_NPOINT // 128):
        far, cx, cy, cz, ai, ax, ay, az = lax.fori_loop(
            0, 128, body, (far, cx, cy, cz, zi, zf, zf, zf), unroll=2)
        sl = pl.ds(j * 128, 128)
        idx_ref[:, sl] = ai
        cx_ref[:, sl] = ax
        cy_ref[:, sl] = ay
        cz_ref[:, sl] = az


def _fps_call(xs, ys, zs):
    out_shape = [
        jax.ShapeDtypeStruct((_B, _NPOINT), jnp.int32),
        jax.ShapeDtypeStruct((_B, _NPOINT), jnp.float32),
        jax.ShapeDtypeStruct((_B, _NPOINT), jnp.float32),
        jax.ShapeDtypeStruct((_B, _NPOINT), jnp.float32),
    ]
    return pl.pallas_call(
        _fps_body,
        out_shape=out_shape,
        scratch_shapes=[pltpu.VMEM((_B, _N), jnp.float32)],
    )(xs, ys, zs)


def _sc_gather(table, idx3):
    info = plsc.get_sparse_core_info()
    nw = info.num_cores * info.num_subcores
    rows_per_w = (_B * _NPOINT) // nw
    chunks = rows_per_w // 128
    mesh = plsc.VectorSubcoreMesh(core_axis_name="c", subcore_axis_name="s")

    @functools.partial(
        pl.kernel,
        mesh=mesh,
        out_type=jax.ShapeDtypeStruct((_B * _NPOINT, _C), jnp.float32),
        scratch_types=[
            pltpu.VMEM((chunks, 128), jnp.int32),
            pltpu.VMEM((rows_per_w, _C), jnp.float32),
            pltpu.SemaphoreType.DMA,
        ],
    )
    def k(table_hbm, idx_hbm, out_hbm, idx_v, rows_v, sem):
        wid = lax.axis_index("s") * info.num_cores + lax.axis_index("c")
        base = wid * rows_per_w
        pltpu.sync_copy(idx_hbm.at[wid], idx_v)
        cps = [
            pltpu.async_copy(table_hbm.at[idx_v.at[c]],
                             rows_v.at[pl.ds(c * 128, 128)], sem)
            for c in range(chunks)
        ]
        for cp in cps:
            cp.wait()
        pltpu.sync_copy(rows_v, out_hbm.at[pl.ds(base, rows_per_w)])

    return k(table, idx3)


def kernel(x, x_xyz, n_pints):
    xs = x_xyz[:, :, 0]
    ys = x_xyz[:, :, 1]
    zs = x_xyz[:, :, 2]
    idxg, cx, cy, cz = _fps_call(xs, ys, zs)
    new_xyz = jnp.stack([cx, cy, cz], axis=-1)

    info = plsc.get_sparse_core_info()
    nw = info.num_cores * info.num_subcores
    chunks = (_B * _NPOINT) // nw // 128
    idx3 = idxg.reshape(nw, chunks, 128)
    new_x = _sc_gather(x.reshape(_B * _N, _C), idx3)
    return (new_x.reshape(_B, _NPOINT, _C), new_xyz)


# speedup vs baseline: 1.6294x; 1.0489x over previous
import functools

import jax
import jax.numpy as jnp
from jax import lax
from jax.experimental import pallas as pl
from jax.experimental.pallas import tpu as pltpu
from jax.experimental.pallas import tpu_sc as plsc

_B = 8
_N = 8192
_C = 128
_NPOINT = 2048


def _fps_body(xs_ref, ys_ref, zs_ref, idx_ref, cx_ref, cy_ref, cz_ref,
              dist_ref):
    lane = lax.broadcasted_iota(jnp.int32, (_B, 128), 1)
    rowbase = lax.broadcasted_iota(jnp.int32, (_B, 1), 0) * _N
    dist_ref[...] = jnp.full((_B, _N), 1e10, jnp.float32)
    nchunk = _N // 128

    def body(l, carry):
        far, cx, cy, cz, ai, ax, ay, az = carry
        lm = lane == l
        ai = jnp.where(lm, far + rowbase, ai)
        ax = jnp.where(lm, cx, ax)
        ay = jnp.where(lm, cy, ay)
        az = jnp.where(lm, cz, az)
        nset = 1
        cps = nchunk // nset
        ninf = jnp.full((_B, 128), -jnp.inf, jnp.float32)
        zi128 = jnp.zeros((_B, 128), jnp.int32)
        zf128 = jnp.zeros((_B, 128), jnp.float32)
        sets = [[ninf, zi128, zf128, zf128, zf128] for _ in range(nset)]
        for s in range(nset):
            sb, si, sx, sy, sz = sets[s]
            for k in range(cps):
                c = s * cps + k
                sl = pl.ds(c * 128, 128)
                xc = xs_ref[:, sl]
                yc = ys_ref[:, sl]
                zc = zs_ref[:, sl]
                d = ((xc - cx) ** 2 + (zc - cz) ** 2) + (yc - cy) ** 2
                dn = jnp.minimum(dist_ref[:, sl], d)
                dist_ref[:, sl] = dn
                g = dn > sb
                sb = jnp.maximum(dn, sb)
                si = jnp.where(g, c, si)
                sx = jnp.where(g, xc, sx)
                sy = jnp.where(g, yc, sy)
                sz = jnp.where(g, zc, sz)
            sets[s] = [sb, si, sx, sy, sz]
        best, besti, bx, by, bz = sets[0]
        for s in range(1, nset):
            sb, si, sx, sy, sz = sets[s]
            g = sb > best
            best = jnp.where(g, sb, best)
            besti = jnp.where(g, si, besti)
            bx = jnp.where(g, sx, bx)
            by = jnp.where(g, sy, by)
            bz = jnp.where(g, sz, bz)
        maxv = jnp.max(best, axis=1, keepdims=True)
        cand = jnp.where(best == maxv, besti * 128 + lane, _N)
        far = jnp.min(cand, axis=1, keepdims=True)
        m2 = cand == far
        cx = jnp.sum(jnp.where(m2, bx, 0.0), axis=1, keepdims=True)
        cy = jnp.sum(jnp.where(m2, by, 0.0), axis=1, keepdims=True)
        cz = jnp.sum(jnp.where(m2, bz, 0.0), axis=1, keepdims=True)
        return far, cx, cy, cz, ai, ax, ay, az

    far = jnp.zeros((_B, 1), jnp.int32)
    cx = xs_ref[:, pl.ds(0, 1)]
    cy = ys_ref[:, pl.ds(0, 1)]
    cz = zs_ref[:, pl.ds(0, 1)]
    zi = jnp.zeros((_B, 128), jnp.int32)
    zf = jnp.zeros((_B, 128), jnp.float32)
    for j in range(_NPOINT // 128):
        far, cx, cy, cz, ai, ax, ay, az = lax.fori_loop(
            0, 128, body, (far, cx, cy, cz, zi, zf, zf, zf), unroll=4)
        sl = pl.ds(j * 128, 128)
        idx_ref[:, sl] = ai
        cx_ref[:, sl] = ax
        cy_ref[:, sl] = ay
        cz_ref[:, sl] = az


def _fps_call(xs, ys, zs):
    out_shape = [
        jax.ShapeDtypeStruct((_B, _NPOINT), jnp.int32),
        jax.ShapeDtypeStruct((_B, _NPOINT), jnp.float32),
        jax.ShapeDtypeStruct((_B, _NPOINT), jnp.float32),
        jax.ShapeDtypeStruct((_B, _NPOINT), jnp.float32),
    ]
    return pl.pallas_call(
        _fps_body,
        out_shape=out_shape,
        scratch_shapes=[pltpu.VMEM((_B, _N), jnp.float32)],
    )(xs, ys, zs)


def _sc_gather(table, idx3):
    info = plsc.get_sparse_core_info()
    nw = info.num_cores * info.num_subcores
    rows_per_w = (_B * _NPOINT) // nw
    chunks = rows_per_w // 128
    mesh = plsc.VectorSubcoreMesh(core_axis_name="c", subcore_axis_name="s")

    @functools.partial(
        pl.kernel,
        mesh=mesh,
        out_type=jax.ShapeDtypeStruct((_B * _NPOINT, _C), jnp.float32),
        scratch_types=[
            pltpu.VMEM((chunks, 128), jnp.int32),
            pltpu.VMEM((rows_per_w, _C), jnp.float32),
            pltpu.SemaphoreType.DMA,
        ],
    )
    def k(table_hbm, idx_hbm, out_hbm, idx_v, rows_v, sem):
        wid = lax.axis_index("s") * info.num_cores + lax.axis_index("c")
        base = wid * rows_per_w
        pltpu.sync_copy(idx_hbm.at[wid], idx_v)
        cps = [
            pltpu.async_copy(table_hbm.at[idx_v.at[c]],
                             rows_v.at[pl.ds(c * 128, 128)], sem)
            for c in range(chunks)
        ]
        for cp in cps:
            cp.wait()
        pltpu.sync_copy(rows_v, out_hbm.at[pl.ds(base, rows_per_w)])

    return k(table, idx3)


def kernel(x, x_xyz, n_pints):
    xs = x_xyz[:, :, 0]
    ys = x_xyz[:, :, 1]
    zs = x_xyz[:, :, 2]
    idxg, cx, cy, cz = _fps_call(xs, ys, zs)
    new_xyz = jnp.stack([cx, cy, cz], axis=-1)

    info = plsc.get_sparse_core_info()
    nw = info.num_cores * info.num_subcores
    chunks = (_B * _NPOINT) // nw // 128
    idx3 = idxg.reshape(nw, chunks, 128)
    new_x = _sc_gather(x.reshape(_B * _N, _C), idx3)
    return (new_x.reshape(_B, _NPOINT, _C), new_xyz)


# speedup vs baseline: 1.6650x; 1.0218x over previous
import functools

import jax
import jax.numpy as jnp
from jax import lax
from jax.experimental import pallas as pl
from jax.experimental.pallas import tpu as pltpu
from jax.experimental.pallas import tpu_sc as plsc

_B = 8
_N = 8192
_C = 128
_NPOINT = 2048


def _fps_body(xs_ref, ys_ref, zs_ref, idx_ref, cx_ref, cy_ref, cz_ref,
              dist_ref):
    lane = lax.broadcasted_iota(jnp.int32, (_B, 128), 1)
    rowbase = lax.broadcasted_iota(jnp.int32, (_B, 1), 0) * _N
    dist_ref[...] = jnp.full((_B, _N), 1e10, jnp.float32)
    nchunk = _N // 128

    def body(l, carry):
        far, cx, cy, cz, ai, ax, ay, az = carry
        lm = lane == l
        ai = jnp.where(lm, far + rowbase, ai)
        ax = jnp.where(lm, cx, ax)
        ay = jnp.where(lm, cy, ay)
        az = jnp.where(lm, cz, az)
        nset = 1
        cps = nchunk // nset
        ninf = jnp.full((_B, 128), -jnp.inf, jnp.float32)
        zi128 = jnp.zeros((_B, 128), jnp.int32)
        zf128 = jnp.zeros((_B, 128), jnp.float32)
        sets = [[ninf, zi128, zf128, zf128, zf128] for _ in range(nset)]
        for s in range(nset):
            sb, si, sx, sy, sz = sets[s]
            for k in range(cps):
                c = s * cps + k
                sl = pl.ds(c * 128, 128)
                xc = xs_ref[:, sl]
                yc = ys_ref[:, sl]
                zc = zs_ref[:, sl]
                d = ((xc - cx) ** 2 + (zc - cz) ** 2) + (yc - cy) ** 2
                dn = jnp.minimum(dist_ref[:, sl], d)
                dist_ref[:, sl] = dn
                g = dn > sb
                sb = jnp.maximum(dn, sb)
                si = jnp.where(g, c, si)
                sx = jnp.where(g, xc, sx)
                sy = jnp.where(g, yc, sy)
                sz = jnp.where(g, zc, sz)
            sets[s] = [sb, si, sx, sy, sz]
        best, besti, bx, by, bz = sets[0]
        for s in range(1, nset):
            sb, si, sx, sy, sz = sets[s]
            g = sb > best
            best = jnp.where(g, sb, best)
            besti = jnp.where(g, si, besti)
            bx = jnp.where(g, sx, bx)
            by = jnp.where(g, sy, by)
            bz = jnp.where(g, sz, bz)
        maxv = jnp.max(best, axis=1, keepdims=True)
        cand = jnp.where(best == maxv, besti * 128 + lane, _N)
        far = jnp.min(cand, axis=1, keepdims=True)
        m2 = cand == far
        cx = jnp.sum(jnp.where(m2, bx, 0.0), axis=1, keepdims=True)
        cy = jnp.sum(jnp.where(m2, by, 0.0), axis=1, keepdims=True)
        cz = jnp.sum(jnp.where(m2, bz, 0.0), axis=1, keepdims=True)
        return far, cx, cy, cz, ai, ax, ay, az

    far = jnp.zeros((_B, 1), jnp.int32)
    cx = xs_ref[:, pl.ds(0, 1)]
    cy = ys_ref[:, pl.ds(0, 1)]
    cz = zs_ref[:, pl.ds(0, 1)]
    zi = jnp.zeros((_B, 128), jnp.int32)
    zf = jnp.zeros((_B, 128), jnp.float32)
    for j in range(_NPOINT // 128):
        far, cx, cy, cz, ai, ax, ay, az = lax.fori_loop(
            0, 128, body, (far, cx, cy, cz, zi, zf, zf, zf), unroll=8)
        sl = pl.ds(j * 128, 128)
        idx_ref[:, sl] = ai
        cx_ref[:, sl] = ax
        cy_ref[:, sl] = ay
        cz_ref[:, sl] = az


def _fps_call(xs, ys, zs):
    out_shape = [
        jax.ShapeDtypeStruct((_B, _NPOINT), jnp.int32),
        jax.ShapeDtypeStruct((_B, _NPOINT), jnp.float32),
        jax.ShapeDtypeStruct((_B, _NPOINT), jnp.float32),
        jax.ShapeDtypeStruct((_B, _NPOINT), jnp.float32),
    ]
    return pl.pallas_call(
        _fps_body,
        out_shape=out_shape,
        scratch_shapes=[pltpu.VMEM((_B, _N), jnp.float32)],
    )(xs, ys, zs)


def _sc_gather(table, idx3):
    info = plsc.get_sparse_core_info()
    nw = info.num_cores * info.num_subcores
    rows_per_w = (_B * _NPOINT) // nw
    chunks = rows_per_w // 128
    mesh = plsc.VectorSubcoreMesh(core_axis_name="c", subcore_axis_name="s")

    @functools.partial(
        pl.kernel,
        mesh=mesh,
        out_type=jax.ShapeDtypeStruct((_B * _NPOINT, _C), jnp.float32),
        scratch_types=[
            pltpu.VMEM((chunks, 128), jnp.int32),
            pltpu.VMEM((rows_per_w, _C), jnp.float32),
            pltpu.SemaphoreType.DMA,
        ],
    )
    def k(table_hbm, idx_hbm, out_hbm, idx_v, rows_v, sem):
        wid = lax.axis_index("s") * info.num_cores + lax.axis_index("c")
        base = wid * rows_per_w
        pltpu.sync_copy(idx_hbm.at[wid], idx_v)
        cps = [
            pltpu.async_copy(table_hbm.at[idx_v.at[c]],
                             rows_v.at[pl.ds(c * 128, 128)], sem)
            for c in range(chunks)
        ]
        for cp in cps:
            cp.wait()
        pltpu.sync_copy(rows_v, out_hbm.at[pl.ds(base, rows_per_w)])

    return k(table, idx3)


def kernel(x, x_xyz, n_pints):
    xs = x_xyz[:, :, 0]
    ys = x_xyz[:, :, 1]
    zs = x_xyz[:, :, 2]
    idxg, cx, cy, cz = _fps_call(xs, ys, zs)
    new_xyz = jnp.stack([cx, cy, cz], axis=-1)

    info = plsc.get_sparse_core_info()
    nw = info.num_cores * info.num_subcores
    chunks = (_B * _NPOINT) // nw // 128
    idx3 = idxg.reshape(nw, chunks, 128)
    new_x = _sc_gather(x.reshape(_B * _N, _C), idx3)
    return (new_x.reshape(_B, _NPOINT, _C), new_xyz)


# speedup vs baseline: 1.6988x; 1.0203x over previous
import functools

import jax
import jax.numpy as jnp
from jax import lax
from jax.experimental import pallas as pl
from jax.experimental.pallas import tpu as pltpu
from jax.experimental.pallas import tpu_sc as plsc

_B = 8
_N = 8192
_C = 128
_NPOINT = 2048


def _fps_body(xs_ref, ys_ref, zs_ref, idx_ref, cx_ref, cy_ref, cz_ref,
              dist_ref):
    lane = lax.broadcasted_iota(jnp.int32, (_B, 128), 1)
    rowbase = lax.broadcasted_iota(jnp.int32, (_B, 1), 0) * _N
    dist_ref[...] = jnp.full((_B, _N), 1e10, jnp.float32)
    nchunk = _N // 128

    def body(l, carry):
        far, cx, cy, cz, ai, ax, ay, az = carry
        lm = lane == l
        ai = jnp.where(lm, far + rowbase, ai)
        ax = jnp.where(lm, cx, ax)
        ay = jnp.where(lm, cy, ay)
        az = jnp.where(lm, cz, az)
        nset = 1
        cps = nchunk // nset
        ninf = jnp.full((_B, 128), -jnp.inf, jnp.float32)
        zi128 = jnp.zeros((_B, 128), jnp.int32)
        zf128 = jnp.zeros((_B, 128), jnp.float32)
        sets = [[ninf, zi128, zf128, zf128, zf128] for _ in range(nset)]
        for s in range(nset):
            sb, si, sx, sy, sz = sets[s]
            for k in range(cps):
                c = s * cps + k
                sl = pl.ds(c * 128, 128)
                xc = xs_ref[:, sl]
                yc = ys_ref[:, sl]
                zc = zs_ref[:, sl]
                d = ((xc - cx) ** 2 + (zc - cz) ** 2) + (yc - cy) ** 2
                dn = jnp.minimum(dist_ref[:, sl], d)
                dist_ref[:, sl] = dn
                g = dn > sb
                sb = jnp.maximum(dn, sb)
                si = jnp.where(g, c, si)
                sx = jnp.where(g, xc, sx)
                sy = jnp.where(g, yc, sy)
                sz = jnp.where(g, zc, sz)
            sets[s] = [sb, si, sx, sy, sz]
        best, besti, bx, by, bz = sets[0]
        for s in range(1, nset):
            sb, si, sx, sy, sz = sets[s]
            g = sb > best
            best = jnp.where(g, sb, best)
            besti = jnp.where(g, si, besti)
            bx = jnp.where(g, sx, bx)
            by = jnp.where(g, sy, by)
            bz = jnp.where(g, sz, bz)
        maxv = jnp.max(best, axis=1, keepdims=True)
        cand = jnp.where(best == maxv, besti * 128 + lane, _N)
        far = jnp.min(cand, axis=1, keepdims=True)
        m2 = cand == far
        cx = jnp.sum(jnp.where(m2, bx, 0.0), axis=1, keepdims=True)
        cy = jnp.sum(jnp.where(m2, by, 0.0), axis=1, keepdims=True)
        cz = jnp.sum(jnp.where(m2, bz, 0.0), axis=1, keepdims=True)
        return far, cx, cy, cz, ai, ax, ay, az

    far = jnp.zeros((_B, 1), jnp.int32)
    cx = xs_ref[:, pl.ds(0, 1)]
    cy = ys_ref[:, pl.ds(0, 1)]
    cz = zs_ref[:, pl.ds(0, 1)]
    zi = jnp.zeros((_B, 128), jnp.int32)
    zf = jnp.zeros((_B, 128), jnp.float32)
    for j in range(_NPOINT // 128):
        far, cx, cy, cz, ai, ax, ay, az = lax.fori_loop(
            0, 128, body, (far, cx, cy, cz, zi, zf, zf, zf), unroll=16)
        sl = pl.ds(j * 128, 128)
        idx_ref[:, sl] = ai
        cx_ref[:, sl] = ax
        cy_ref[:, sl] = ay
        cz_ref[:, sl] = az


def _fps_call(xs, ys, zs):
    out_shape = [
        jax.ShapeDtypeStruct((_B, _NPOINT), jnp.int32),
        jax.ShapeDtypeStruct((_B, _NPOINT), jnp.float32),
        jax.ShapeDtypeStruct((_B, _NPOINT), jnp.float32),
        jax.ShapeDtypeStruct((_B, _NPOINT), jnp.float32),
    ]
    return pl.pallas_call(
        _fps_body,
        out_shape=out_shape,
        scratch_shapes=[pltpu.VMEM((_B, _N), jnp.float32)],
    )(xs, ys, zs)


def _sc_gather(table, idx3):
    info = plsc.get_sparse_core_info()
    nw = info.num_cores * info.num_subcores
    rows_per_w = (_B * _NPOINT) // nw
    chunks = rows_per_w // 128
    mesh = plsc.VectorSubcoreMesh(core_axis_name="c", subcore_axis_name="s")

    @functools.partial(
        pl.kernel,
        mesh=mesh,
        out_type=jax.ShapeDtypeStruct((_B * _NPOINT, _C), jnp.float32),
        scratch_types=[
            pltpu.VMEM((chunks, 128), jnp.int32),
            pltpu.VMEM((rows_per_w, _C), jnp.float32),
            pltpu.SemaphoreType.DMA,
        ],
    )
    def k(table_hbm, idx_hbm, out_hbm, idx_v, rows_v, sem):
        wid = lax.axis_index("s") * info.num_cores + lax.axis_index("c")
        base = wid * rows_per_w
        pltpu.sync_copy(idx_hbm.at[wid], idx_v)
        cps = [
            pltpu.async_copy(table_hbm.at[idx_v.at[c]],
                             rows_v.at[pl.ds(c * 128, 128)], sem)
            for c in range(chunks)
        ]
        for cp in cps:
            cp.wait()
        pltpu.sync_copy(rows_v, out_hbm.at[pl.ds(base, rows_per_w)])

    return k(table, idx3)


def kernel(x, x_xyz, n_pints):
    xs = x_xyz[:, :, 0]
    ys = x_xyz[:, :, 1]
    zs = x_xyz[:, :, 2]
    idxg, cx, cy, cz = _fps_call(xs, ys, zs)
    new_xyz = jnp.stack([cx, cy, cz], axis=-1)

    info = plsc.get_sparse_core_info()
    nw = info.num_cores * info.num_subcores
    chunks = (_B * _NPOINT) // nw // 128
    idx3 = idxg.reshape(nw, chunks, 128)
    new_x = _sc_gather(x.reshape(_B * _N, _C), idx3)
    return (new_x.reshape(_B, _NPOINT, _C), new_xyz)
